# baseline (speedup 1.0000x reference)
import jax
import jax.numpy as jnp
from jax import lax
from jax.experimental import pallas as pl
from jax.experimental.pallas import tpu as pltpu
from jax.experimental.pallas import tpu_sc as plsc

_T = 11
_EX = 8
_SQ = 512
_L = 16
_NC = 2


_NW = 32


def _sc_probs_body(histT_hbm, w16_hbm, sm_hbm, aux_hbm, out_hbm,
                   h_v, w_v, part_v, comb_v, sm_v, aux_v, pT_v, shared_v):
    wid = lax.axis_index("s") * _NC + lax.axis_index("c")
    hid = histT_hbm.shape[0]
    rows = hid // _NW

    pltpu.sync_copy(histT_hbm.at[pl.ds(wid * rows, rows)], h_v)
    pltpu.sync_copy(w16_hbm.at[pl.ds(wid * rows, rows)], w_v)
    acc = jnp.zeros((_L,), jnp.float32)
    for k in range(rows):
        acc = acc + h_v[k] * w_v[k]
    part_v[0] = acc
    pltpu.sync_copy(part_v, shared_v.at[pl.ds(wid, 1)])
    plsc.subcore_barrier()

    @pl.when(wid == 0)
    def _():
        pltpu.sync_copy(sm_hbm, sm_v)
        pltpu.sync_copy(aux_hbm, aux_v)
        pltpu.sync_copy(shared_v, comb_v)
        logit = jnp.zeros((_L,), jnp.float32)
        for wkr in range(_NW):
            logit = logit + comb_v[wkr]
        logit = logit + aux_v[pl.ds(0, _L)]
        length = sm_v[...].astype(jnp.float32)
        lane = lax.broadcasted_iota(jnp.int32, (_L,), 0).astype(jnp.float32)
        num_vec = aux_v[pl.ds(_L, _L)]
        one = jnp.full((_L,), 1.0, jnp.float32)
        zero = jnp.zeros((_L,), jnp.float32)
        rowm = jnp.where(num_vec > lane, one, zero)
        b_vec = aux_v[pl.ds(0, _L)]
        denom = jnp.zeros((_L,), jnp.float32)
        for turn in range(_T):
            mask = jnp.where(length >= float(_T - turn), one, zero)
            lrow = logit if turn == _T - 1 else b_vec
            e_t = jnp.exp(lrow) * mask * rowm
            pT_v[turn] = e_t
            denom = denom + e_t
        for turn in range(_T):
            pT_v[turn] = pT_v[turn] / denom
        for turn in range(_T, _L):
            pT_v[turn] = jnp.zeros((_L,), jnp.float32)
        pltpu.sync_copy(pT_v, out_hbm)


def _sc_probs(histT, w16, sm, aux):
    hid, bs = histT.shape
    return pl.kernel(
        _sc_probs_body,
        out_type=jax.ShapeDtypeStruct((_L, bs), jnp.float32),
        mesh=plsc.VectorSubcoreMesh(core_axis_name="c", subcore_axis_name="s"),
        scratch_types=[
            pltpu.VMEM((hid // _NW, _L), jnp.float32),
            pltpu.VMEM((hid // _NW, _L), jnp.float32),
            pltpu.VMEM((1, _L), jnp.float32),
            pltpu.VMEM((_NW, _L), jnp.float32),
            pltpu.VMEM((_L,), jnp.int32),
            pltpu.VMEM((2 * _L,), jnp.float32),
            pltpu.VMEM((_L, _L), jnp.float32),
            pltpu.VMEM_SHARED((_NW, _L), jnp.float32),
        ],
    )(histT, w16, sm, aux)


def _scale_kernel(num_ref, b_ref, sm_ref, hist_ref, mtl_ref, wt_ref,
                  bert_ref, nbert_ref, nmtl_ref):
    i = pl.program_id(0)
    bs = hist_ref.shape[1]
    w = wt_ref[0, :]
    h = hist_ref[0]
    bias = b_ref[0]
    logit = jnp.sum(h * w[None, :], axis=1) + bias
    t = lax.broadcasted_iota(jnp.int32, (bs, _T), 1)
    r = lax.broadcasted_iota(jnp.int32, (bs, _T), 0)
    lengths = sm_ref[0][:, None]
    mask = (t >= _T - lengths).astype(jnp.float32)
    rowm = (r < num_ref[0]).astype(jnp.float32)
    lrow = jnp.where(t == _T - 1, logit[:, None], bias)
    e = jnp.exp(lrow) * mask * rowm
    p = e / jnp.sum(e, axis=1, keepdims=True)
    s = p[:, _T - 1]
    nmtl_ref[0] = mtl_ref[0] * s[:, None]
    col = lax.broadcasted_iota(jnp.int32, (_EX, bs), 1)
    row = lax.broadcasted_iota(jnp.int32, (_EX, bs), 0)
    sel = (col == i * _EX + row).astype(jnp.float32)
    sblk = jnp.sum(sel * s[None, :], axis=1)
    nbert_ref[...] = bert_ref[...] * sblk[:, None, None]


def kernel(bert_representation, history_attention_input, mtl_input,
           slice_mask, slice_num, W, b):
    bs, seq, hid = bert_representation.shape
    wt = W.reshape(1, hid)
    sm32 = slice_mask.astype(jnp.int32)
    num = jnp.asarray(slice_num, jnp.int32).reshape(1)

    aux = jnp.concatenate([
        jnp.broadcast_to(b.astype(jnp.float32), (_L,)),
        jnp.broadcast_to(num.astype(jnp.float32), (_L,)),
    ])
    probs_t = _sc_probs(history_attention_input.T,
                        jnp.broadcast_to(W, (hid, bs)), sm32, aux)

    grid = (bs // _EX, seq // _SQ)
    nbert, nmtl = pl.pallas_call(
        _scale_kernel,
        grid=grid,
        in_specs=[
            pl.BlockSpec(memory_space=pltpu.SMEM),
            pl.BlockSpec(memory_space=pltpu.SMEM),
            pl.BlockSpec((1, bs), lambda i, j: (0, 0)),
            pl.BlockSpec((1, bs, hid), lambda i, j: (0, 0, 0)),
            pl.BlockSpec((1, bs, hid), lambda i, j: (0, 0, 0)),
            pl.BlockSpec((1, hid), lambda i, j: (0, 0)),
            pl.BlockSpec((_EX, _SQ, hid), lambda i, j: (i, j, 0)),
        ],
        out_specs=[
            pl.BlockSpec((_EX, _SQ, hid), lambda i, j: (i, j, 0)),
            pl.BlockSpec((1, bs, hid), lambda i, j: (0, 0, 0)),
        ],
        out_shape=[
            jax.ShapeDtypeStruct((bs, seq, hid), jnp.float32),
            jax.ShapeDtypeStruct((1, bs, hid), jnp.float32),
        ],
        compiler_params=pltpu.CompilerParams(
            dimension_semantics=("parallel", "parallel"),
        ),
    )(num, b, sm32.reshape(1, bs),
      history_attention_input.reshape(1, bs, hid),
      mtl_input.reshape(1, bs, hid), wt, bert_representation)
    return nbert, nmtl.reshape(bs, hid), probs_t.T[:, :_T]


# speedup vs baseline: 1.0445x; 1.0445x over previous
import jax
import jax.numpy as jnp
from jax import lax
from jax.experimental import pallas as pl
from jax.experimental.pallas import tpu as pltpu
from jax.experimental.pallas import tpu_sc as plsc

_T = 11
_EX = 8
_SQ = 512
_L = 16
_NC = 2


_NW = 16


def _sc_probs_body(histT_hbm, w16_hbm, sm_hbm, aux_hbm, out_hbm,
                   h_v, w_v, part_v, comb_v, sm_v, aux_v, pT_v, shared_v):
    wid = lax.axis_index("s")
    hid = histT_hbm.shape[0]
    rows = hid // _NW

    pltpu.sync_copy(histT_hbm.at[pl.ds(wid * rows, rows)], h_v)
    pltpu.sync_copy(w16_hbm.at[pl.ds(wid * rows, rows)], w_v)
    acc = jnp.zeros((_L,), jnp.float32)
    for k in range(rows):
        acc = acc + h_v[k] * w_v[k]
    part_v[0] = acc
    pltpu.sync_copy(part_v, shared_v.at[pl.ds(wid, 1)])
    plsc.subcore_barrier()

    @pl.when(wid == 0)
    def _():
        pltpu.sync_copy(sm_hbm, sm_v)
        pltpu.sync_copy(aux_hbm, aux_v)
        pltpu.sync_copy(shared_v, comb_v)
        logit = jnp.zeros((_L,), jnp.float32)
        for wkr in range(_NW):
            logit = logit + comb_v[wkr]
        logit = logit + aux_v[pl.ds(0, _L)]
        length = sm_v[...].astype(jnp.float32)
        lane = lax.broadcasted_iota(jnp.int32, (_L,), 0).astype(jnp.float32)
        num_vec = aux_v[pl.ds(_L, _L)]
        one = jnp.full((_L,), 1.0, jnp.float32)
        zero = jnp.zeros((_L,), jnp.float32)
        rowm = jnp.where(num_vec > lane, one, zero)
        b_vec = aux_v[pl.ds(0, _L)]
        denom = jnp.zeros((_L,), jnp.float32)
        for turn in range(_T):
            mask = jnp.where(length >= float(_T - turn), one, zero)
            lrow = logit if turn == _T - 1 else b_vec
            e_t = jnp.exp(lrow) * mask * rowm
            pT_v[turn] = e_t
            denom = denom + e_t
        for turn in range(_T):
            pT_v[turn] = pT_v[turn] / denom
        for turn in range(_T, _L):
            pT_v[turn] = jnp.zeros((_L,), jnp.float32)
        pltpu.sync_copy(pT_v, out_hbm)


def _sc_probs(histT, w16, sm, aux):
    hid, bs = histT.shape
    return pl.kernel(
        _sc_probs_body,
        out_type=jax.ShapeDtypeStruct((_L, bs), jnp.float32),
        mesh=plsc.VectorSubcoreMesh(core_axis_name="c", subcore_axis_name="s",
                                    num_cores=1),
        scratch_types=[
            pltpu.VMEM((hid // _NW, _L), jnp.float32),
            pltpu.VMEM((hid // _NW, _L), jnp.float32),
            pltpu.VMEM((1, _L), jnp.float32),
            pltpu.VMEM((_NW, _L), jnp.float32),
            pltpu.VMEM((_L,), jnp.int32),
            pltpu.VMEM((2 * _L,), jnp.float32),
            pltpu.VMEM((_L, _L), jnp.float32),
            pltpu.VMEM_SHARED((_NW, _L), jnp.float32),
        ],
    )(histT, w16, sm, aux)


def _scale_kernel(num_ref, b_ref, sm_ref, hist_ref, mtl_ref, wt_ref,
                  bert_ref, nbert_ref, nmtl_ref):
    i = pl.program_id(0)
    bs = hist_ref.shape[1]
    w = wt_ref[0, :]
    h = hist_ref[0]
    bias = b_ref[0]
    logit = jnp.sum(h * w[None, :], axis=1) + bias
    t = lax.broadcasted_iota(jnp.int32, (bs, _T), 1)
    r = lax.broadcasted_iota(jnp.int32, (bs, _T), 0)
    lengths = sm_ref[0][:, None]
    mask = (t >= _T - lengths).astype(jnp.float32)
    rowm = (r < num_ref[0]).astype(jnp.float32)
    lrow = jnp.where(t == _T - 1, logit[:, None], bias)
    e = jnp.exp(lrow) * mask * rowm
    p = e / jnp.sum(e, axis=1, keepdims=True)
    s = p[:, _T - 1]
    nmtl_ref[0] = mtl_ref[0] * s[:, None]
    col = lax.broadcasted_iota(jnp.int32, (_EX, bs), 1)
    row = lax.broadcasted_iota(jnp.int32, (_EX, bs), 0)
    sel = (col == i * _EX + row).astype(jnp.float32)
    sblk = jnp.sum(sel * s[None, :], axis=1)
    nbert_ref[...] = bert_ref[...] * sblk[:, None, None]


def kernel(bert_representation, history_attention_input, mtl_input,
           slice_mask, slice_num, W, b):
    bs, seq, hid = bert_representation.shape
    wt = W.reshape(1, hid)
    sm32 = slice_mask.astype(jnp.int32)
    num = jnp.asarray(slice_num, jnp.int32).reshape(1)

    aux = jnp.concatenate([
        jnp.broadcast_to(b.astype(jnp.float32), (_L,)),
        jnp.broadcast_to(num.astype(jnp.float32), (_L,)),
    ])
    probs_t = _sc_probs(history_attention_input.T,
                        jnp.broadcast_to(W, (hid, bs)), sm32, aux)

    grid = (bs // _EX, seq // _SQ)
    nbert, nmtl = pl.pallas_call(
        _scale_kernel,
        grid=grid,
        in_specs=[
            pl.BlockSpec(memory_space=pltpu.SMEM),
            pl.BlockSpec(memory_space=pltpu.SMEM),
            pl.BlockSpec((1, bs), lambda i, j: (0, 0)),
            pl.BlockSpec((1, bs, hid), lambda i, j: (0, 0, 0)),
            pl.BlockSpec((1, bs, hid), lambda i, j: (0, 0, 0)),
            pl.BlockSpec((1, hid), lambda i, j: (0, 0)),
            pl.BlockSpec((_EX, _SQ, hid), lambda i, j: (i, j, 0)),
        ],
        out_specs=[
            pl.BlockSpec((_EX, _SQ, hid), lambda i, j: (i, j, 0)),
            pl.BlockSpec((1, bs, hid), lambda i, j: (0, 0, 0)),
        ],
        out_shape=[
            jax.ShapeDtypeStruct((bs, seq, hid), jnp.float32),
            jax.ShapeDtypeStruct((1, bs, hid), jnp.float32),
        ],
        compiler_params=pltpu.CompilerParams(
            dimension_semantics=("parallel", "parallel"),
        ),
    )(num, b, sm32.reshape(1, bs),
      history_attention_input.reshape(1, bs, hid),
      mtl_input.reshape(1, bs, hid), wt, bert_representation)
    return nbert, nmtl.reshape(bs, hid), probs_t.T[:, :_T]


# speedup vs baseline: 1.1158x; 1.0682x over previous
import jax
import jax.numpy as jnp
from jax import lax
from jax.experimental import pallas as pl
from jax.experimental.pallas import tpu as pltpu
from jax.experimental.pallas import tpu_sc as plsc

_T = 11
_EX = 8
_SQ = 512
_L = 16
_NC = 2


_NW = 16


def _sc_probs_body(hist_hbm, w_hbm, aux_hbm, out_hbm,
                   h_v, w_v, aux_v, p_v, sem_h, sem_w, sem_a):
    wid = lax.axis_index("s")
    hid = hist_hbm.shape[1]

    cp_h = pltpu.async_copy(hist_hbm.at[pl.ds(wid, 1)], h_v, sem_h)
    cp_w = pltpu.async_copy(w_hbm, w_v, sem_w)
    cp_a = pltpu.async_copy(aux_hbm, aux_v, sem_a)
    cp_h.wait()
    cp_w.wait()
    acc = jnp.zeros((_L,), jnp.float32)
    for c in range(hid // _L):
        acc = acc + h_v[0, pl.ds(c * _L, _L)] * w_v[pl.ds(c * _L, _L)]
    logit = acc[0]
    for k in range(1, _L):
        logit = logit + acc[k]
    cp_a.wait()
    length = aux_v[pl.ds(wid, _L)][0]
    b_vec = aux_v[pl.ds(_L, _L)]
    num_vec = aux_v[pl.ds(2 * _L, _L)]
    t = lax.broadcasted_iota(jnp.int32, (_L,), 0)
    tf = t.astype(jnp.float32)
    one = jnp.full((_L,), 1.0, jnp.float32)
    zero = jnp.zeros((_L,), jnp.float32)
    mask = jnp.where(tf >= (zero + float(_T) - length), one, zero)
    mask = jnp.where(tf < float(_T), mask, zero)
    widf = (t * 0 + wid).astype(jnp.float32)
    rowm = jnp.where(num_vec > widf, one, zero)
    sel = jnp.where(t == _T - 1, one, zero)
    e = jnp.exp(b_vec + sel * (zero + logit)) * mask * rowm
    denom = e[0]
    for k in range(1, _T):
        denom = denom + e[k]
    p_v[0] = e / (zero + denom)
    pltpu.sync_copy(p_v, out_hbm.at[pl.ds(wid, 1)])


def _sc_probs(hist, w_vec, aux):
    bs, hid = hist.shape
    return pl.kernel(
        _sc_probs_body,
        out_type=jax.ShapeDtypeStruct((bs, _L), jnp.float32),
        mesh=plsc.VectorSubcoreMesh(core_axis_name="c", subcore_axis_name="s",
                                    num_cores=1),
        scratch_types=[
            pltpu.VMEM((1, hid), jnp.float32),
            pltpu.VMEM((hid,), jnp.float32),
            pltpu.VMEM((3 * _L,), jnp.float32),
            pltpu.VMEM((1, _L), jnp.float32),
            pltpu.SemaphoreType.DMA,
            pltpu.SemaphoreType.DMA,
            pltpu.SemaphoreType.DMA,
        ],
    )(hist, w_vec, aux)


def _scale_kernel(num_ref, b_ref, sm_ref, hist_ref, mtl_ref, wt_ref,
                  bert_ref, nbert_ref, nmtl_ref):
    i = pl.program_id(0)
    bs = hist_ref.shape[1]
    w = wt_ref[0, :]
    h = hist_ref[0]
    bias = b_ref[0]
    logit = jnp.sum(h * w[None, :], axis=1) + bias
    t = lax.broadcasted_iota(jnp.int32, (bs, _T), 1)
    r = lax.broadcasted_iota(jnp.int32, (bs, _T), 0)
    lengths = sm_ref[0][:, None]
    mask = (t >= _T - lengths).astype(jnp.float32)
    rowm = (r < num_ref[0]).astype(jnp.float32)
    lrow = jnp.where(t == _T - 1, logit[:, None], bias)
    e = jnp.exp(lrow) * mask * rowm
    p = e / jnp.sum(e, axis=1, keepdims=True)
    s = p[:, _T - 1]
    nmtl_ref[0] = mtl_ref[0] * s[:, None]
    col = lax.broadcasted_iota(jnp.int32, (_EX, bs), 1)
    row = lax.broadcasted_iota(jnp.int32, (_EX, bs), 0)
    sel = (col == i * _EX + row).astype(jnp.float32)
    sblk = jnp.sum(sel * s[None, :], axis=1)
    nbert_ref[...] = bert_ref[...] * sblk[:, None, None]


def kernel(bert_representation, history_attention_input, mtl_input,
           slice_mask, slice_num, W, b):
    bs, seq, hid = bert_representation.shape
    wt = W.reshape(1, hid)
    sm32 = slice_mask.astype(jnp.int32)
    num = jnp.asarray(slice_num, jnp.int32).reshape(1)

    aux = jnp.concatenate([
        sm32.astype(jnp.float32),
        jnp.broadcast_to(b.astype(jnp.float32), (_L,)),
        jnp.broadcast_to(num.astype(jnp.float32), (_L,)),
    ])
    probs_pad = _sc_probs(history_attention_input, W.reshape(hid), aux)

    grid = (bs // _EX, seq // _SQ)
    nbert, nmtl = pl.pallas_call(
        _scale_kernel,
        grid=grid,
        in_specs=[
            pl.BlockSpec(memory_space=pltpu.SMEM),
            pl.BlockSpec(memory_space=pltpu.SMEM),
            pl.BlockSpec((1, bs), lambda i, j: (0, 0)),
            pl.BlockSpec((1, bs, hid), lambda i, j: (0, 0, 0)),
            pl.BlockSpec((1, bs, hid), lambda i, j: (0, 0, 0)),
            pl.BlockSpec((1, hid), lambda i, j: (0, 0)),
            pl.BlockSpec((_EX, _SQ, hid), lambda i, j: (i, j, 0)),
        ],
        out_specs=[
            pl.BlockSpec((_EX, _SQ, hid), lambda i, j: (i, j, 0)),
            pl.BlockSpec((1, bs, hid), lambda i, j: (0, 0, 0)),
        ],
        out_shape=[
            jax.ShapeDtypeStruct((bs, seq, hid), jnp.float32),
            jax.ShapeDtypeStruct((1, bs, hid), jnp.float32),
        ],
        compiler_params=pltpu.CompilerParams(
            dimension_semantics=("parallel", "parallel"),
        ),
    )(num, b, sm32.reshape(1, bs),
      history_attention_input.reshape(1, bs, hid),
      mtl_input.reshape(1, bs, hid), wt, bert_representation)
    return nbert, nmtl.reshape(bs, hid), probs_pad[:, :_T]
